# baseline (speedup 1.0000x reference)
import jax
import jax.numpy as jnp
from jax import lax
from jax.experimental import pallas as pl
from jax.experimental.pallas import tpu as pltpu
from jax.experimental.pallas import tpu_sc as plsc

TOTAL_GRAPHS = 100000
STOP = 100000
D = 128
BATCH = 16384

NC = 2
NS = 16
NW = NC * NS
BPW = BATCH // NW
LANES = 16
CHUNKS = BPW // LANES


def _body(idx_hbm, table_hbm, stop_hbm, out_hbm, idx_v, safe_v, rows_v,
          idx_s, pos_s, cnt_s, sem):
    wid = lax.axis_index("s") * NC + lax.axis_index("c")
    base = wid * BPW

    pltpu.sync_copy(idx_hbm.at[pl.ds(base, BPW)], idx_v)

    for i in range(CHUNKS):
        v = idx_v[pl.ds(i * LANES, LANES)]
        safe_v[pl.ds(i * LANES, LANES)] = jnp.where(v == STOP, 0, v)
    gather = pltpu.async_copy(table_hbm.at[safe_v], rows_v, sem)

    acc = jnp.zeros((LANES,), jnp.int32)
    for i in range(CHUNKS):
        v = idx_v[pl.ds(i * LANES, LANES)]
        for j in range(LANES):
            idx_s[i * LANES + j] = v[j]
        acc = acc + jnp.where(v == STOP, 1, 0)
    have_stops = plsc.all_reduce_population_count(acc > 0)[0]

    cnt_s[0] = 0

    def scan_row(r, carry):
        @pl.when(idx_s[r] == STOP)
        def _():
            c = cnt_s[0]
            pos_s[c] = r
            cnt_s[0] = c + 1
        return carry

    lax.fori_loop(0, BPW, scan_row, 0, unroll=8)

    gather.wait()

    cnt = cnt_s[0]

    def patch_group(g, carry):
        @pl.when(cnt > g * LANES)
        def _():
            for j in range(LANES):
                p = g * LANES + j

                @pl.when(p < cnt)
                def _():
                    pltpu.sync_copy(stop_hbm, rows_v.at[pos_s[p]])
        return carry

    lax.fori_loop(0, CHUNKS, patch_group, 0)

    pltpu.sync_copy(rows_v, out_hbm.at[pl.ds(base, BPW)])


@jax.jit
def _gather(idx, table, stop):
    mesh = plsc.VectorSubcoreMesh(core_axis_name="c", subcore_axis_name="s",
                                  num_cores=NC, num_subcores=NS)
    return pl.kernel(
        _body,
        out_type=jax.ShapeDtypeStruct((BATCH, D), jnp.float32),
        mesh=mesh,
        scratch_types=[
            pltpu.VMEM((BPW,), jnp.int32),
            pltpu.VMEM((BPW,), jnp.int32),
            pltpu.VMEM((BPW, D), jnp.float32),
            pltpu.SMEM((BPW,), jnp.int32),
            pltpu.SMEM((BPW,), jnp.int32),
            pltpu.SMEM((8,), jnp.int32),
            pltpu.SemaphoreType.DMA,
        ],
    )(idx, table, stop)


def kernel(symbol_tensor_in, graph_table, stop_embedding):
    return _gather(symbol_tensor_in.astype(jnp.int32), graph_table,
                   stop_embedding)


# speedup vs baseline: 1.0454x; 1.0454x over previous
import jax
import jax.numpy as jnp
from jax import lax
from jax.experimental import pallas as pl
from jax.experimental.pallas import tpu as pltpu
from jax.experimental.pallas import tpu_sc as plsc

TOTAL_GRAPHS = 100000
STOP = 100000
D = 128
BATCH = 16384

NC = 2
NS = 16
NW = NC * NS
BPW = BATCH // NW
LANES = 16
CHUNKS = BPW // LANES


def _body(idx_hbm, table_hbm, stop_hbm, out_hbm, idx_v, safe_v, rows_v,
          bits_s, pos_s, cnt_s, sem):
    wid = lax.axis_index("s") * NC + lax.axis_index("c")
    base = wid * BPW

    pltpu.sync_copy(idx_hbm.at[pl.ds(base, BPW)], idx_v)

    for i in range(CHUNKS):
        v = idx_v[pl.ds(i * LANES, LANES)]
        safe_v[pl.ds(i * LANES, LANES)] = jnp.where(v == STOP, 0, v)
    gather = pltpu.async_copy(table_hbm.at[safe_v], rows_v, sem)

    lane = lax.iota(jnp.int32, LANES)
    perms = [lane ^ st for st in (8, 4, 2, 1)]
    for i in range(CHUNKS):
        v = idx_v[pl.ds(i * LANES, LANES)]
        bitv = jnp.where(v == STOP, jnp.int32(1) << lane, 0)
        for p in perms:
            bitv = bitv + bitv[p]
        bits_s[i] = bitv[0]

    cnt_s[0] = 0

    def scan_chunk(g, carry):
        b = bits_s[g]

        @pl.when(b != 0)
        def _():
            for j in range(LANES):
                @pl.when((b >> j) & 1 != 0)
                def _():
                    c = cnt_s[0]
                    pos_s[c] = g * LANES + j
                    cnt_s[0] = c + 1
        return carry

    lax.fori_loop(0, CHUNKS, scan_chunk, 0)

    gather.wait()

    cnt = cnt_s[0]

    def patch_group(g, carry):
        @pl.when(cnt > g * LANES)
        def _():
            for j in range(LANES):
                p = g * LANES + j

                @pl.when(p < cnt)
                def _():
                    pltpu.sync_copy(stop_hbm, rows_v.at[pos_s[p]])
        return carry

    lax.fori_loop(0, CHUNKS, patch_group, 0)

    pltpu.sync_copy(rows_v, out_hbm.at[pl.ds(base, BPW)])


@jax.jit
def _gather(idx, table, stop):
    mesh = plsc.VectorSubcoreMesh(core_axis_name="c", subcore_axis_name="s",
                                  num_cores=NC, num_subcores=NS)
    return pl.kernel(
        _body,
        out_type=jax.ShapeDtypeStruct((BATCH, D), jnp.float32),
        mesh=mesh,
        scratch_types=[
            pltpu.VMEM((BPW,), jnp.int32),
            pltpu.VMEM((BPW,), jnp.int32),
            pltpu.VMEM((BPW, D), jnp.float32),
            pltpu.SMEM((CHUNKS,), jnp.int32),
            pltpu.SMEM((BPW,), jnp.int32),
            pltpu.SMEM((8,), jnp.int32),
            pltpu.SemaphoreType.DMA,
        ],
    )(idx, table, stop)


def kernel(symbol_tensor_in, graph_table, stop_embedding):
    return _gather(symbol_tensor_in.astype(jnp.int32), graph_table,
                   stop_embedding)
